# baseline (speedup 1.0000x reference)
import jax
import jax.numpy as jnp
from jax import lax
from jax.experimental import pallas as pl
from jax.experimental.pallas import tpu as pltpu
from jax.experimental.pallas import tpu_sc as plsc

_CYCLE = 168
_LEN = 336
_CH = 256
_B = 1024
_NC = 2
_NS = 16
_NW = _NC * _NS
_SC_B = 256
_BPW = _SC_B // _NW


def _sc_body(idx_hbm, data_hbm, out_hbm, idx_v, ddd_v, sem):
    cid = lax.axis_index("c")
    sid = lax.axis_index("s")
    wid = sid * _NC + cid
    base = wid * _BPW
    pltpu.sync_copy(idx_hbm.at[pl.ds(base, _BPW)], idx_v.at[pl.ds(0, _BPW)])
    pltpu.sync_copy(data_hbm, ddd_v.at[pl.ds(0, _CYCLE)])
    pltpu.sync_copy(data_hbm, ddd_v.at[pl.ds(_CYCLE, _CYCLE)])
    pltpu.sync_copy(data_hbm, ddd_v.at[pl.ds(2 * _CYCLE, _CYCLE)])
    copies = []
    vec = idx_v[pl.ds(0, 16)]
    for b in range(_BPW):
        i = vec[b]
        copies.append(
            pltpu.async_copy(ddd_v.at[pl.ds(i, _LEN)], out_hbm.at[base + b], sem)
        )
    for c in copies:
        c.wait()


def _sc_kernel(start, data):
    mesh = plsc.VectorSubcoreMesh(core_axis_name="c", subcore_axis_name="s")
    k = pl.kernel(
        _sc_body,
        out_type=jax.ShapeDtypeStruct((_SC_B, _LEN, _CH), jnp.float32),
        mesh=mesh,
        scratch_types=[
            pltpu.VMEM((16,), jnp.int32),
            pltpu.VMEM((3 * _CYCLE, _CH), jnp.float32),
            pltpu.SemaphoreType.DMA,
        ],
        compiler_params=pltpu.CompilerParams(use_tc_tiling_on_sc=False),
    )
    return k(start, data)


def _tc_body(s_ref, data_ref, out_ref, quad, rots):
    pid = pl.program_id(0)

    @pl.when(pid == 0)
    def _build():
        for k in range(4):
            quad[pl.ds(k * _CYCLE, _CYCLE), :] = data_ref[...]
        for r in range(8):
            rots[r] = quad[pl.ds(r, 3 * _CYCLE), :]

    for k in range(4):
        i = s_ref[pid * 4 + k]
        r = lax.rem(i, 8)
        off = pl.multiple_of(i - r, 8)
        out_ref[k] = rots[r, pl.ds(off, _LEN), :]


def _tc_kernel(start, data):
    nb = start.shape[0]
    grid_spec = pltpu.PrefetchScalarGridSpec(
        num_scalar_prefetch=1,
        grid=(nb // 4,),
        in_specs=[pl.BlockSpec((_CYCLE, _CH), lambda b, s: (0, 0))],
        out_specs=pl.BlockSpec((4, _LEN, _CH), lambda b, s: (b, 0, 0)),
        scratch_shapes=[
            pltpu.VMEM((4 * _CYCLE, _CH), jnp.float32),
            pltpu.VMEM((8, 3 * _CYCLE, _CH), jnp.float32),
        ],
    )
    return pl.pallas_call(
        _tc_body,
        grid_spec=grid_spec,
        out_shape=jax.ShapeDtypeStruct((nb, _LEN, _CH), jnp.float32),
    )(start, data)


def kernel(index, length, data):
    start = jnp.mod(index.astype(jnp.int32) + (length - _LEN), _CYCLE)
    start = start.astype(jnp.int32)
    sc_part = _sc_kernel(start[:_SC_B], data)
    tc_part = _tc_kernel(start[_SC_B:], data)
    return jnp.concatenate([sc_part, tc_part], axis=0)
